# baseline (speedup 1.0000x reference)
import functools

import jax
import jax.numpy as jnp
from jax import lax
from jax.experimental import pallas as pl
from jax.experimental.pallas import tpu as pltpu
from jax.experimental.pallas import tpu_sc as plsc

F32 = jnp.float32


def _bdot(x, w):
    return jnp.dot(x.astype(jnp.bfloat16), w.astype(jnp.bfloat16),
                   preferred_element_type=F32)


def _dense_body(x_ref, w_ref, o_ref):
    o_ref[...] = _bdot(x_ref[...], w_ref[...])


def _dense_bias_relu_body(x_ref, w_ref, b_ref, o_ref):
    y = _bdot(x_ref[...], w_ref[...])
    o_ref[...] = jnp.maximum(y + b_ref[...], 0.0)


def _matmul(x, w, block):
    m, d = x.shape
    dout = w.shape[1]
    return pl.pallas_call(
        _dense_body,
        grid=(m // block,),
        in_specs=[
            pl.BlockSpec((block, d), lambda i: (i, 0)),
            pl.BlockSpec((d, dout), lambda i: (0, 0)),
        ],
        out_specs=pl.BlockSpec((block, dout), lambda i: (i, 0)),
        out_shape=jax.ShapeDtypeStruct((m, dout), F32),
    )(x, w)


def _affine_relu(x, w, b, block):
    m, d = x.shape
    dout = w.shape[1]
    return pl.pallas_call(
        _dense_bias_relu_body,
        grid=(m // block,),
        in_specs=[
            pl.BlockSpec((block, d), lambda i: (i, 0)),
            pl.BlockSpec((d, dout), lambda i: (0, 0)),
            pl.BlockSpec((1, dout), lambda i: (0, 0)),
        ],
        out_specs=pl.BlockSpec((block, dout), lambda i: (i, 0)),
        out_shape=jax.ShapeDtypeStruct((m, dout), F32),
    )(x, w, b.reshape(1, dout))


_L = 64
_NS = 16


def _sc_segment(table, esrc, edst, zeros, eh=None):
    n, d = table.shape
    with_mul = eh is not None
    e = esrc.shape[0]
    per_w = e // (2 * _NS)
    full, tail = divmod(per_w, _L)
    assert full % 2 == 0 and full >= 4 and tail % 8 == 0
    npad = zeros.shape[0]
    rows_per_tile = npad // _NS

    mesh = plsc.VectorSubcoreMesh(core_axis_name="core",
                                  subcore_axis_name="subcore")

    scratch = [
        pltpu.VMEM_SHARED((npad, d), F32),
        pltpu.VMEM((_L,), jnp.int32),
        pltpu.VMEM((_L,), jnp.int32),
        pltpu.VMEM((_L,), jnp.int32),
        pltpu.VMEM((_L,), jnp.int32),
        pltpu.VMEM((_L, 128), F32),
        pltpu.VMEM((_L, 128), F32),
        pltpu.VMEM((_L, 128), F32),
        pltpu.VMEM((_L, 128), F32),
        pltpu.VMEM((max(tail, 8),), jnp.int32),
        pltpu.VMEM((max(tail, 8),), jnp.int32),
        pltpu.SemaphoreType.DMA((2,)),
        pltpu.SemaphoreType.DMA((2,)),
        pltpu.SemaphoreType.DMA((2,)),
        pltpu.SemaphoreType.DMA((2,)),
    ]

    @functools.partial(
        pl.kernel,
        out_type=(jax.ShapeDtypeStruct((npad, d), F32),
                  jax.ShapeDtypeStruct((npad, d), F32)),
        mesh=mesh,
        scratch_types=scratch,
    )
    def sc_kernel(src_hbm, esrc_hbm, edst_hbm, zeros_hbm, eh_hbm,
                  out0, out1,
                  acc, is0, is1, id0, id1, g0, g1, e0b, e1b,
                  tidx_s, tidx_d, semi, semg, seme, sems):
        c = lax.axis_index("core")
        s = lax.axis_index("subcore")
        r0 = s * rows_per_tile
        IS = (is0, is1)
        ID = (id0, id1)
        G = (g0, g1)
        EB = (e0b, e1b)

        pltpu.sync_copy(zeros_hbm.at[pl.ds(r0, rows_per_tile)],
                        acc.at[pl.ds(r0, rows_per_tile)])

        def run_edges(base, nfull, tail):
            def issue_idx(k, b):
                o = base + k * _L
                pltpu.async_copy(esrc_hbm.at[pl.ds(o, _L)], IS[b],
                                 semi.at[b])
                pltpu.async_copy(edst_hbm.at[pl.ds(o, _L)], ID[b],
                                 semi.at[b])

            def wait_idx(b):
                pltpu.make_async_copy(esrc_hbm.at[pl.ds(0, _L)], IS[b],
                                      semi.at[b]).wait()
                pltpu.make_async_copy(edst_hbm.at[pl.ds(0, _L)], ID[b],
                                      semi.at[b]).wait()

            def issue_fetch(k, b):
                pltpu.async_copy(src_hbm.at[IS[b]], G[b], semg.at[b])
                if with_mul:
                    o = base + k * _L
                    pltpu.async_copy(eh_hbm.at[pl.ds(o, _L)], EB[b],
                                     seme.at[b])

            def wait_fetch(b):
                pltpu.make_async_copy(src_hbm.at[IS[b]], G[b],
                                      semg.at[b]).wait()
                if with_mul:
                    pltpu.make_async_copy(eh_hbm.at[pl.ds(0, _L)], EB[b],
                                          seme.at[b]).wait()

            def mul(b):
                if with_mul:
                    @pl.loop(0, _L)
                    def _(i):
                        for q in range(d // 16):
                            sl = pl.ds(q * 16, 16)
                            G[b][i, sl] = G[b][i, sl] * EB[b][i, sl]

            def issue_scatter(b):
                pltpu.async_copy(G[b], acc.at[ID[b]], sems.at[b],
                                 add=True)

            def wait_scatter(b):
                pltpu.make_async_copy(G[b], acc.at[ID[b]],
                                      sems.at[b]).wait()

            issue_idx(0, 0)
            issue_idx(1, 1)
            wait_idx(0)
            issue_fetch(0, 0)
            plsc.subcore_barrier()

            def chunk_step(k, b, last):
                wait_fetch(b)
                mul(b)
                issue_scatter(b)
                if not last:
                    wait_scatter(b)
                    issue_idx(k + 2, b)
                    wait_idx(b)
                    issue_fetch(k + 2, b)

            wait_idx(1)
            issue_fetch(1, 1)
            @pl.loop(0, nfull // 2 - 1)
            def _(j):
                k = j * 2
                chunk_step(k, 0, False)
                chunk_step(k + 1, 1, False)

            chunk_step(nfull - 2, 0, True)
            chunk_step(nfull - 1, 1, True)
            wait_scatter(0)
            wait_scatter(1)

            if tail:
                o = base + nfull * _L
                pltpu.sync_copy(esrc_hbm.at[pl.ds(o, tail)], tidx_s)
                pltpu.sync_copy(edst_hbm.at[pl.ds(o, tail)], tidx_d)
                pltpu.async_copy(src_hbm.at[tidx_s],
                                 G[0].at[pl.ds(0, tail)],
                                 semg.at[0]).wait()
                if with_mul:
                    pltpu.sync_copy(eh_hbm.at[pl.ds(o, tail)],
                                    EB[0].at[pl.ds(0, tail)])

                    @pl.loop(0, tail)
                    def _(i):
                        for q in range(d // 16):
                            sl = pl.ds(q * 16, 16)
                            G[0][i, sl] = G[0][i, sl] * EB[0][i, sl]

                pltpu.sync_copy(G[0].at[pl.ds(0, tail)],
                                acc.at[tidx_d], add=True)

        w = c * _NS + s
        run_edges(w * per_w, full, tail)

        plsc.subcore_barrier()

        @pl.when(c == 0)
        def _out0():
            pltpu.sync_copy(acc.at[pl.ds(r0, rows_per_tile)],
                            out0.at[pl.ds(r0, rows_per_tile)])

        @pl.when(c == 1)
        def _out1():
            pltpu.sync_copy(acc.at[pl.ds(r0, rows_per_tile)],
                            out1.at[pl.ds(r0, rows_per_tile)])

    if eh is None:
        eh = jnp.zeros((8, d), F32)
    return sc_kernel(table, esrc, edst, zeros, eh)


def _node_update_body(hd_ref, md0_ref, md1_ref, mf0_ref, mf1_ref,
                      wdst_ref, bdst_ref,
                      wnu_ref, bnu_ref, iw1_ref, ib1_ref, iw2_ref, ib2_ref,
                      aw1_ref, ab1_ref, aw2_ref, ab2_ref, o_ref):
    def mm(a, b):
        return jnp.dot(a, b, preferred_element_type=F32)

    x = hd_ref[...]
    d = x.shape[1]
    dst = jnp.maximum(mm(x, wdst_ref[...]) + bdst_ref[...], 0.0)
    wnu = wnu_ref[...]
    m = mm(dst, wnu[0:d]) \
        + mm(md0_ref[...] + md1_ref[...], wnu[d:2 * d]) \
        + mm(mf0_ref[...] + mf1_ref[...], wnu[2 * d:3 * d])
    m = jnp.maximum(m + bnu_ref[...], 0.0)
    t = jnp.maximum(mm(m, iw1_ref[...]) + ib1_ref[...], 0.0)
    m = m + jnp.maximum(mm(t, iw2_ref[...]) + ib2_ref[...], 0.0)
    h = x + m
    t2 = jnp.maximum(mm(h, aw1_ref[...]) + ab1_ref[...], 0.0)
    o_ref[...] = h + jnp.maximum(mm(t2, aw2_ref[...]) + ab2_ref[...], 0.0)


def _node_update(h_d, md0, md1, mf0, mf1, W_dst, b_dst, W_nu, b_nu,
                 ir_W1, ir_b1, ir_W2, ir_b2, ar_W1, ar_b1, ar_W2, ar_b2,
                 block=2000):
    n, d = h_d.shape
    row = lambda i: (i, 0)
    fixw = lambda i: (0, 0)
    wspec = pl.BlockSpec((d, d), fixw)
    bspec = pl.BlockSpec((1, d), fixw)
    mspec = pl.BlockSpec((block, d), row)
    return pl.pallas_call(
        _node_update_body,
        grid=(n // block,),
        in_specs=[
            mspec, mspec, mspec, mspec, mspec,
            wspec, bspec,
            pl.BlockSpec((3 * d, d), fixw), bspec,
            wspec, bspec, wspec, bspec,
            wspec, bspec, wspec, bspec,
        ],
        out_specs=pl.BlockSpec((block, d), row),
        out_shape=jax.ShapeDtypeStruct((n, d), F32),
    )(h_d, md0, md1, mf0, mf1, W_dst, b_dst.reshape(1, d), W_nu,
      b_nu.reshape(1, d), ir_W1, ir_b1.reshape(1, d), ir_W2,
      ir_b2.reshape(1, d), ar_W1, ar_b1.reshape(1, d), ar_W2,
      ar_b2.reshape(1, d))


def kernel(node_feat_domestic, edge_feat, node_feat_foreign, a2a_edge_index,
           b2a_src, b2a_dst, W_G, W_sd, b_sd, W_sf, b_sf, W_dst, b_dst,
           W_nu, b_nu, ir_W1, ir_b1, ir_W2, ir_b2,
           ar_W1, ar_b1, ar_W2, ar_b2):
    n, d = node_feat_domestic.shape
    npad = ((n + 8 * _NS - 1) // (8 * _NS)) * (8 * _NS)
    zeros = jnp.zeros((npad, d), F32)
    src_hf = _affine_relu(node_feat_foreign, W_sf, b_sf, block=1280)
    mf0, mf1 = _sc_segment(src_hf, b2a_src, b2a_dst, zeros)
    eh = _matmul(edge_feat, W_G, block=1280)
    src_h = _affine_relu(node_feat_domestic, W_sd, b_sd, block=2000)
    md0, md1 = _sc_segment(src_h, a2a_edge_index[0], a2a_edge_index[1],
                           zeros, eh=eh)
    return _node_update(node_feat_domestic, md0, md1, mf0, mf1,
                        W_dst, b_dst, W_nu, b_nu, ir_W1, ir_b1,
                        ir_W2, ir_b2, ar_W1, ar_b1, ar_W2, ar_b2)


# speedup vs baseline: 1.0918x; 1.0918x over previous
import functools

import jax
import jax.numpy as jnp
from jax import lax
from jax.experimental import pallas as pl
from jax.experimental.pallas import tpu as pltpu
from jax.experimental.pallas import tpu_sc as plsc

F32 = jnp.float32


def _bdot(x, w):
    return jnp.dot(x.astype(jnp.bfloat16), w.astype(jnp.bfloat16),
                   preferred_element_type=F32)


def _dense_body(x_ref, w_ref, o_ref):
    o_ref[...] = _bdot(x_ref[...], w_ref[...]).astype(o_ref.dtype)


def _dense_bias_relu_body(x_ref, w_ref, b_ref, o_ref):
    y = _bdot(x_ref[...], w_ref[...])
    o_ref[...] = jnp.maximum(y + b_ref[...], 0.0)


def _matmul(x, w, block, out_dtype=F32):
    m, d = x.shape
    dout = w.shape[1]
    return pl.pallas_call(
        _dense_body,
        grid=(m // block,),
        in_specs=[
            pl.BlockSpec((block, d), lambda i: (i, 0)),
            pl.BlockSpec((d, dout), lambda i: (0, 0)),
        ],
        out_specs=pl.BlockSpec((block, dout), lambda i: (i, 0)),
        out_shape=jax.ShapeDtypeStruct((m, dout), out_dtype),
    )(x, w)


def _affine_relu(x, w, b, block):
    m, d = x.shape
    dout = w.shape[1]
    return pl.pallas_call(
        _dense_bias_relu_body,
        grid=(m // block,),
        in_specs=[
            pl.BlockSpec((block, d), lambda i: (i, 0)),
            pl.BlockSpec((d, dout), lambda i: (0, 0)),
            pl.BlockSpec((1, dout), lambda i: (0, 0)),
        ],
        out_specs=pl.BlockSpec((block, dout), lambda i: (i, 0)),
        out_shape=jax.ShapeDtypeStruct((m, dout), F32),
    )(x, w, b.reshape(1, dout))


_L = 64
_NS = 16


def _sc_segment(table, esrc, edst, zeros, eh=None):
    n, d = table.shape
    with_mul = eh is not None
    e = esrc.shape[0]
    per_w = e // (2 * _NS)
    full, tail = divmod(per_w, _L)
    assert full % 2 == 0 and full >= 4 and tail % 8 == 0
    npad = zeros.shape[0]
    rows_per_tile = npad // _NS

    mesh = plsc.VectorSubcoreMesh(core_axis_name="core",
                                  subcore_axis_name="subcore")

    scratch = [
        pltpu.VMEM_SHARED((npad, d), F32),
        pltpu.VMEM((_L,), jnp.int32),
        pltpu.VMEM((_L,), jnp.int32),
        pltpu.VMEM((_L,), jnp.int32),
        pltpu.VMEM((_L,), jnp.int32),
        pltpu.VMEM((_L, 128), F32),
        pltpu.VMEM((_L, 128), F32),
        pltpu.VMEM((_L, 128), F32),
        pltpu.VMEM((_L, 128), F32),
        pltpu.VMEM((max(tail, 8),), jnp.int32),
        pltpu.VMEM((max(tail, 8),), jnp.int32),
        pltpu.SemaphoreType.DMA((2,)),
        pltpu.SemaphoreType.DMA((2,)),
        pltpu.SemaphoreType.DMA((2,)),
        pltpu.SemaphoreType.DMA((2,)),
    ]

    @functools.partial(
        pl.kernel,
        out_type=(jax.ShapeDtypeStruct((npad, d), F32),
                  jax.ShapeDtypeStruct((npad, d), F32)),
        mesh=mesh,
        scratch_types=scratch,
    )
    def sc_kernel(src_hbm, esrc_hbm, edst_hbm, zeros_hbm, eh_hbm,
                  out0, out1,
                  acc, is0, is1, id0, id1, g0, g1, e0b, e1b,
                  tidx_s, tidx_d, semi, semg, seme, sems):
        c = lax.axis_index("core")
        s = lax.axis_index("subcore")
        r0 = s * rows_per_tile
        IS = (is0, is1)
        ID = (id0, id1)
        G = (g0, g1)
        EB = (e0b, e1b)

        pltpu.sync_copy(zeros_hbm.at[pl.ds(r0, rows_per_tile)],
                        acc.at[pl.ds(r0, rows_per_tile)])

        def run_edges(base, nfull, tail):
            def issue_idx(k, b):
                o = base + k * _L
                pltpu.async_copy(esrc_hbm.at[pl.ds(o, _L)], IS[b],
                                 semi.at[b])
                pltpu.async_copy(edst_hbm.at[pl.ds(o, _L)], ID[b],
                                 semi.at[b])

            def wait_idx(b):
                pltpu.make_async_copy(esrc_hbm.at[pl.ds(0, _L)], IS[b],
                                      semi.at[b]).wait()
                pltpu.make_async_copy(edst_hbm.at[pl.ds(0, _L)], ID[b],
                                      semi.at[b]).wait()

            def issue_fetch(k, b):
                pltpu.async_copy(src_hbm.at[IS[b]], G[b], semg.at[b])
                if with_mul:
                    o = base + k * _L
                    pltpu.async_copy(eh_hbm.at[pl.ds(o, _L)], EB[b],
                                     seme.at[b])

            def wait_fetch(b):
                pltpu.make_async_copy(src_hbm.at[IS[b]], G[b],
                                      semg.at[b]).wait()
                if with_mul:
                    pltpu.make_async_copy(eh_hbm.at[pl.ds(0, _L)], EB[b],
                                          seme.at[b]).wait()

            def mul_row(gbuf, ebuf, i):
                for q in range(d // 16):
                    sl = pl.ds(q * 16, 16)
                    gbuf[i, sl] = gbuf[i, sl] * ebuf[i, sl]

            def mul(b):
                if with_mul:
                    @pl.loop(0, _L)
                    def _(i):
                        mul_row(G[b], EB[b], i)

            def issue_scatter(b):
                pltpu.async_copy(G[b], acc.at[ID[b]], sems.at[b],
                                 add=True)

            def wait_scatter(b):
                pltpu.make_async_copy(G[b], acc.at[ID[b]],
                                      sems.at[b]).wait()

            issue_idx(0, 0)
            issue_idx(1, 1)
            wait_idx(0)
            issue_fetch(0, 0)
            plsc.subcore_barrier()

            def chunk_step(k, b, last):
                wait_fetch(b)
                mul(b)
                issue_scatter(b)
                if not last:
                    wait_scatter(b)
                    issue_idx(k + 2, b)
                    wait_idx(b)
                    issue_fetch(k + 2, b)

            wait_idx(1)
            issue_fetch(1, 1)
            @pl.loop(0, nfull // 2 - 1)
            def _(j):
                k = j * 2
                chunk_step(k, 0, False)
                chunk_step(k + 1, 1, False)

            chunk_step(nfull - 2, 0, True)
            chunk_step(nfull - 1, 1, True)
            wait_scatter(0)
            wait_scatter(1)

            if tail:
                o = base + nfull * _L
                pltpu.sync_copy(esrc_hbm.at[pl.ds(o, tail)], tidx_s)
                pltpu.sync_copy(edst_hbm.at[pl.ds(o, tail)], tidx_d)
                pltpu.async_copy(src_hbm.at[tidx_s],
                                 G[0].at[pl.ds(0, tail)],
                                 semg.at[0]).wait()
                if with_mul:
                    pltpu.sync_copy(eh_hbm.at[pl.ds(o, tail)],
                                    EB[0].at[pl.ds(0, tail)])

                    @pl.loop(0, tail)
                    def _(i):
                        mul_row(G[0], EB[0], i)

                pltpu.sync_copy(G[0].at[pl.ds(0, tail)],
                                acc.at[tidx_d], add=True)

        w = c * _NS + s
        run_edges(w * per_w, full, tail)

        plsc.subcore_barrier()

        @pl.when(c == 0)
        def _out0():
            pltpu.sync_copy(acc.at[pl.ds(r0, rows_per_tile)],
                            out0.at[pl.ds(r0, rows_per_tile)])

        @pl.when(c == 1)
        def _out1():
            pltpu.sync_copy(acc.at[pl.ds(r0, rows_per_tile)],
                            out1.at[pl.ds(r0, rows_per_tile)])

    if eh is None:
        eh = jnp.zeros((8, d), F32)
    return sc_kernel(table, esrc, edst, zeros, eh)


def _node_update_body(hd_ref, md0_ref, md1_ref, mf0_ref, mf1_ref,
                      wdst_ref, bdst_ref,
                      wnu_ref, bnu_ref, iw1_ref, ib1_ref, iw2_ref, ib2_ref,
                      aw1_ref, ab1_ref, aw2_ref, ab2_ref, o_ref):
    def mm(a, b):
        return jnp.dot(a, b, preferred_element_type=F32)

    x = hd_ref[...]
    d = x.shape[1]
    dst = jnp.maximum(mm(x, wdst_ref[...]) + bdst_ref[...], 0.0)
    wnu = wnu_ref[...]
    m = mm(dst, wnu[0:d]) \
        + mm(md0_ref[...] + md1_ref[...], wnu[d:2 * d]) \
        + mm(mf0_ref[...] + mf1_ref[...], wnu[2 * d:3 * d])
    m = jnp.maximum(m + bnu_ref[...], 0.0)
    t = jnp.maximum(mm(m, iw1_ref[...]) + ib1_ref[...], 0.0)
    m = m + jnp.maximum(mm(t, iw2_ref[...]) + ib2_ref[...], 0.0)
    h = x + m
    t2 = jnp.maximum(mm(h, aw1_ref[...]) + ab1_ref[...], 0.0)
    o_ref[...] = h + jnp.maximum(mm(t2, aw2_ref[...]) + ab2_ref[...], 0.0)


def _node_update(h_d, md0, md1, mf0, mf1, W_dst, b_dst, W_nu, b_nu,
                 ir_W1, ir_b1, ir_W2, ir_b2, ar_W1, ar_b1, ar_W2, ar_b2,
                 block=2000):
    n, d = h_d.shape
    row = lambda i: (i, 0)
    fixw = lambda i: (0, 0)
    wspec = pl.BlockSpec((d, d), fixw)
    bspec = pl.BlockSpec((1, d), fixw)
    mspec = pl.BlockSpec((block, d), row)
    return pl.pallas_call(
        _node_update_body,
        grid=(n // block,),
        in_specs=[
            mspec, mspec, mspec, mspec, mspec,
            wspec, bspec,
            pl.BlockSpec((3 * d, d), fixw), bspec,
            wspec, bspec, wspec, bspec,
            wspec, bspec, wspec, bspec,
        ],
        out_specs=pl.BlockSpec((block, d), row),
        out_shape=jax.ShapeDtypeStruct((n, d), F32),
    )(h_d, md0, md1, mf0, mf1, W_dst, b_dst.reshape(1, d), W_nu,
      b_nu.reshape(1, d), ir_W1, ir_b1.reshape(1, d), ir_W2,
      ir_b2.reshape(1, d), ar_W1, ar_b1.reshape(1, d), ar_W2,
      ar_b2.reshape(1, d))


def kernel(node_feat_domestic, edge_feat, node_feat_foreign, a2a_edge_index,
           b2a_src, b2a_dst, W_G, W_sd, b_sd, W_sf, b_sf, W_dst, b_dst,
           W_nu, b_nu, ir_W1, ir_b1, ir_W2, ir_b2,
           ar_W1, ar_b1, ar_W2, ar_b2):
    n, d = node_feat_domestic.shape
    npad = ((n + 8 * _NS - 1) // (8 * _NS)) * (8 * _NS)
    zeros = jnp.zeros((npad, d), F32)
    src_hf = _affine_relu(node_feat_foreign, W_sf, b_sf, block=1280)
    mf0, mf1 = _sc_segment(src_hf, b2a_src, b2a_dst, zeros)
    edge_feat_b, _ = lax.optimization_barrier((edge_feat, src_hf))
    eh = _matmul(edge_feat_b, W_G, block=1280)
    src_h = _affine_relu(node_feat_domestic, W_sd, b_sd, block=2000)
    md0, md1 = _sc_segment(src_h, a2a_edge_index[0], a2a_edge_index[1],
                           zeros, eh=eh)
    return _node_update(node_feat_domestic, md0, md1, mf0, mf1,
                        W_dst, b_dst, W_nu, b_nu, ir_W1, ir_b1,
                        ir_W2, ir_b2, ar_W1, ar_b1, ar_W2, ar_b2)
